# baseline (speedup 1.0000x reference)
import functools

import jax
import jax.numpy as jnp
from jax import lax
from jax.experimental import pallas as pl
from jax.experimental.pallas import tpu as pltpu
from jax.experimental.pallas import tpu_sc as plsc

N = 10000
NPAD = 10240
E = 320000
EPAD = 327680
EROWS = EPAD // 128
D_IN = 128
D_HID = 256
D_OUT = 40
DOP = 64
NC, NS = 2, 16
RPT = NPAD // NS
K1 = EROWS // NS
K2 = EROWS // (NC * NS)
BM = 256
GRID_M = NPAD // BM

_f32 = jnp.float32


def _mesh():
  return plsc.VectorSubcoreMesh(core_axis_name="c", subcore_axis_name="s")


def _sc_deg():
  @functools.partial(
      pl.kernel,
      out_type=jax.ShapeDtypeStruct((2, NPAD, 16), _f32),
      mesh=_mesh(),
      compiler_params=pltpu.CompilerParams(use_tc_tiling_on_sc=False),
      scratch_types=[
          pltpu.VMEM((K2, 128), jnp.int32),
          pltpu.VMEM((128, 16), _f32),
          pltpu.VMEM_SHARED((NPAD, 16), _f32),
      ],
  )
  def deg(dstr, ones, zeros, out, dst_v, ones_v, acc):
    c = lax.axis_index("c")
    s = lax.axis_index("s")
    rows = pl.ds(s * RPT, RPT)
    pltpu.sync_copy(zeros.at[rows], acc.at[rows])
    pltpu.sync_copy(ones, ones_v)
    row0 = (c * NS + s) * K2
    pltpu.sync_copy(dstr.at[pl.ds(row0, K2)], dst_v)
    plsc.subcore_barrier()

    def body(j, carry):
      pltpu.sync_copy(ones_v, acc.at[dst_v.at[j]], add=True)
      return carry

    lax.fori_loop(0, K2, body, 0)
    plsc.subcore_barrier()
    pltpu.sync_copy(acc.at[rows], out.at[c].at[rows])

  return deg


def _sc_agg(D, H, k, feature_split, dtype=_f32, dup_table=False):
  out_slabs = H if feature_split else 2
  passes = H // 2 if feature_split else 1
  NB = 8 if (dtype == jnp.bfloat16 and D <= 64) else 4
  assert k % NB == 0

  @functools.partial(
      pl.kernel,
      out_type=jax.ShapeDtypeStruct((out_slabs, NPAD, D), dtype),
      mesh=_mesh(),
      compiler_params=pltpu.CompilerParams(use_tc_tiling_on_sc=False),
      scratch_types=[
          pltpu.VMEM((k, 128), jnp.int32),
          pltpu.VMEM((k, 128), jnp.int32),
      ] + [pltpu.VMEM((128, D), dtype)] * NB
        + [pltpu.SemaphoreType.DMA] * (2 * NB)
      + [pltpu.VMEM_SHARED((NPAD, D), dtype)],
  )
  def agg(table, srcr, dstr, zeros, out, src_v, dst_v, *rest):
    bufs = rest[:NB]
    semg = rest[NB:2 * NB]
    sems = rest[2 * NB:3 * NB]
    acc = rest[3 * NB]
    c = lax.axis_index("c")
    s = lax.axis_index("s")
    rows = pl.ds(s * RPT, RPT)
    if feature_split:
      row0 = s * k
    else:
      row0 = (c * NS + s) * k
    pltpu.sync_copy(srcr.at[pl.ds(row0, k)], src_v)
    pltpu.sync_copy(dstr.at[pl.ds(row0, k)], dst_v)

    for p in range(passes):
      h = c * passes + p if feature_split else (c if dup_table else 0)
      o = h if feature_split else c
      pltpu.sync_copy(zeros.at[rows], acc.at[rows])
      plsc.subcore_barrier()

      for b in range(NB):
        pltpu.async_copy(table.at[h].at[src_v.at[b]], bufs[b], semg[b])

      def body(t, carry):
        base = t * NB
        for b in range(NB):
          pltpu.make_async_copy(table.at[h].at[src_v.at[base + b]],
                                bufs[b], semg[b]).wait()
          pltpu.async_copy(bufs[b], acc.at[dst_v.at[base + b]], sems[b],
                           add=True)
        for b in range(NB):
          pltpu.make_async_copy(bufs[b], acc.at[dst_v.at[base + b]],
                                sems[b]).wait()
          pltpu.async_copy(table.at[h].at[src_v.at[base + NB + b]],
                           bufs[b], semg[b])
        return carry

      lax.fori_loop(0, k // NB - 1, body, 0)
      base = k - NB
      for b in range(NB):
        pltpu.make_async_copy(table.at[h].at[src_v.at[base + b]],
                              bufs[b], semg[b]).wait()
        pltpu.async_copy(bufs[b], acc.at[dst_v.at[base + b]], sems[b],
                         add=True)
      for b in range(NB):
        pltpu.make_async_copy(bufs[b], acc.at[dst_v.at[base + b]],
                              sems[b]).wait()
      plsc.subcore_barrier()
      pltpu.sync_copy(acc.at[rows], out.at[o].at[rows])

  return agg


def _tc_a(x_ref, w1_ref, deg_ref, g1_ref, dinv_ref):
  deg = deg_ref[0][:, 0:1] + deg_ref[1][:, 0:1] + 1.0
  dinv = lax.rsqrt(deg)
  h = jnp.dot(x_ref[...], w1_ref[...], preferred_element_type=_f32)
  g = (h * dinv).astype(jnp.bfloat16)
  for q in range(2):
    g1_ref[q] = g[:, q * 128:(q + 1) * 128]
  dinv_ref[...] = dinv


def _tc_b(s1_ref, g1_ref, dinv_ref, b1_ref, w2_ref, g2_ref):
  dinv = dinv_ref[...]
  z = jnp.concatenate(
      [s1_ref[q][...].astype(_f32) + g1_ref[q][...].astype(_f32)
       for q in range(2)], axis=1)
  z = jnp.maximum(z * dinv + b1_ref[...], 0.0)
  h2 = jnp.dot(z, w2_ref[...], preferred_element_type=_f32)
  g2 = (h2 * dinv).astype(jnp.bfloat16)
  g2_ref[0] = g2
  g2_ref[1] = g2


def _tc_c(s2_ref, g2_ref, dinv_ref, b2_ref, out_ref):
  z = ((s2_ref[0][...].astype(_f32) + s2_ref[1][...].astype(_f32)
        + g2_ref[0][...].astype(_f32)) * dinv_ref[...] + b2_ref[...])
  mask = lax.broadcasted_iota(jnp.int32, (1, DOP), 1) < D_OUT
  zm = jnp.where(mask, z, -jnp.inf)
  mx = jnp.max(zm, axis=1, keepdims=True)
  ex = jnp.where(mask, jnp.exp(z - mx), 0.0)
  lse = jnp.log(jnp.sum(ex, axis=1, keepdims=True)) + mx
  out_ref[...] = z - lse


def kernel(x, edge_index, W1, b1, W2, b2):
  x_pad = jnp.zeros((NPAD, D_IN), _f32).at[:N].set(x)
  pad = jnp.full((EPAD - E,), N, jnp.int32)
  srcr = jnp.concatenate([edge_index[0], pad]).reshape(EROWS, 128)
  dstr = jnp.concatenate([edge_index[1], pad]).reshape(EROWS, 128)
  w2p = jnp.zeros((D_HID, DOP), _f32).at[:, :D_OUT].set(W2)
  b1r = b1.reshape(1, D_HID)
  b2p = jnp.zeros((1, DOP), _f32).at[0, :D_OUT].set(b2)
  ones16 = jnp.ones((128, 16), _f32)
  z16 = jnp.zeros((NPAD, 16), _f32)

  deg2 = _sc_deg()(dstr, ones16, z16)

  g1, dinv = pl.pallas_call(
      _tc_a,
      grid=(GRID_M,),
      in_specs=[
          pl.BlockSpec((BM, D_IN), lambda m: (m, 0)),
          pl.BlockSpec((D_IN, D_HID), lambda m: (0, 0)),
          pl.BlockSpec((2, BM, 16), lambda m: (0, m, 0)),
      ],
      out_specs=[
          pl.BlockSpec((2, BM, 128), lambda m: (0, m, 0)),
          pl.BlockSpec((BM, 1), lambda m: (m, 0)),
      ],
      out_shape=[
          jax.ShapeDtypeStruct((2, NPAD, 128), jnp.bfloat16),
          jax.ShapeDtypeStruct((NPAD, 1), _f32),
      ],
  )(x_pad, W1, deg2)

  z64b = jnp.zeros((NPAD, 64), jnp.bfloat16)
  z128b = jnp.zeros((NPAD, 128), jnp.bfloat16)
  s1 = _sc_agg(128, 2, K1, True, jnp.bfloat16)(g1, srcr, dstr, z128b)

  g2 = pl.pallas_call(
      _tc_b,
      grid=(GRID_M,),
      in_specs=[
          pl.BlockSpec((2, BM, 128), lambda m: (0, m, 0)),
          pl.BlockSpec((2, BM, 128), lambda m: (0, m, 0)),
          pl.BlockSpec((BM, 1), lambda m: (m, 0)),
          pl.BlockSpec((1, D_HID), lambda m: (0, 0)),
          pl.BlockSpec((D_HID, DOP), lambda m: (0, 0)),
      ],
      out_specs=pl.BlockSpec((2, BM, DOP), lambda m: (0, m, 0)),
      out_shape=jax.ShapeDtypeStruct((2, NPAD, DOP), jnp.bfloat16),
  )(s1, g1, dinv, b1r, w2p)

  s2 = _sc_agg(DOP, 1, K2, False, jnp.bfloat16, dup_table=True)(
      g2, srcr, dstr, z64b)

  out_full = pl.pallas_call(
      _tc_c,
      grid=(GRID_M,),
      in_specs=[
          pl.BlockSpec((2, BM, DOP), lambda m: (0, m, 0)),
          pl.BlockSpec((1, BM, DOP), lambda m: (0, m, 0)),
          pl.BlockSpec((BM, 1), lambda m: (m, 0)),
          pl.BlockSpec((1, DOP), lambda m: (0, 0)),
      ],
      out_specs=pl.BlockSpec((BM, DOP), lambda m: (m, 0)),
      out_shape=jax.ShapeDtypeStruct((NPAD, DOP), _f32),
  )(s2, g2, dinv, b2p)

  return out_full[:N, :D_OUT]


# speedup vs baseline: 1.1626x; 1.1626x over previous
import functools

import jax
import jax.numpy as jnp
from jax import lax
from jax.experimental import pallas as pl
from jax.experimental.pallas import tpu as pltpu
from jax.experimental.pallas import tpu_sc as plsc

N = 10000
NPAD = 10240
E = 320000
EPAD = 327680
EROWS = EPAD // 128
D_IN = 128
D_HID = 256
D_OUT = 40
DOP = 64
NC, NS = 2, 16
RPT = NPAD // NS
K1 = EROWS // NS
K2 = EROWS // (NC * NS)
BM = 512
GRID_M = NPAD // BM

_f32 = jnp.float32


def _mesh():
  return plsc.VectorSubcoreMesh(core_axis_name="c", subcore_axis_name="s")


def _sc_deg():
  @functools.partial(
      pl.kernel,
      out_type=jax.ShapeDtypeStruct((2, NPAD, 16), _f32),
      mesh=_mesh(),
      compiler_params=pltpu.CompilerParams(use_tc_tiling_on_sc=False),
      scratch_types=[
          pltpu.VMEM((K2, 128), jnp.int32),
          pltpu.VMEM((128, 16), _f32),
          pltpu.VMEM_SHARED((NPAD, 16), _f32),
      ],
  )
  def deg(dstr, ones, zeros, out, dst_v, ones_v, acc):
    c = lax.axis_index("c")
    s = lax.axis_index("s")
    rows = pl.ds(s * RPT, RPT)
    pltpu.sync_copy(zeros.at[rows], acc.at[rows])
    pltpu.sync_copy(ones, ones_v)
    row0 = (c * NS + s) * K2
    pltpu.sync_copy(dstr.at[pl.ds(row0, K2)], dst_v)
    plsc.subcore_barrier()

    def body(j, carry):
      pltpu.sync_copy(ones_v, acc.at[dst_v.at[j]], add=True)
      return carry

    lax.fori_loop(0, K2, body, 0)
    plsc.subcore_barrier()
    pltpu.sync_copy(acc.at[rows], out.at[c].at[rows])

  return deg


def _sc_agg(D, H, k, feature_split, dtype=_f32, dup_table=False):
  out_slabs = H if feature_split else 2
  passes = H // 2 if feature_split else 1
  NB = 8 if (dtype == jnp.bfloat16 and D <= 64) else 4
  assert k % NB == 0

  @functools.partial(
      pl.kernel,
      out_type=jax.ShapeDtypeStruct((out_slabs, NPAD, D), dtype),
      mesh=_mesh(),
      compiler_params=pltpu.CompilerParams(use_tc_tiling_on_sc=False),
      scratch_types=[
          pltpu.VMEM((k, 128), jnp.int32),
          pltpu.VMEM((k, 128), jnp.int32),
      ] + [pltpu.VMEM((128, D), dtype)] * NB
        + [pltpu.SemaphoreType.DMA] * (2 * NB)
      + [pltpu.VMEM_SHARED((NPAD, D), dtype)],
  )
  def agg(table, srcr, dstr, zeros, out, src_v, dst_v, *rest):
    bufs = rest[:NB]
    semg = rest[NB:2 * NB]
    sems = rest[2 * NB:3 * NB]
    acc = rest[3 * NB]
    c = lax.axis_index("c")
    s = lax.axis_index("s")
    rows = pl.ds(s * RPT, RPT)
    if feature_split:
      row0 = s * k
    else:
      row0 = (c * NS + s) * k
    pltpu.sync_copy(srcr.at[pl.ds(row0, k)], src_v)
    pltpu.sync_copy(dstr.at[pl.ds(row0, k)], dst_v)

    for p in range(passes):
      h = c * passes + p if feature_split else (c if dup_table else 0)
      o = h if feature_split else c
      pltpu.sync_copy(zeros.at[rows], acc.at[rows])
      plsc.subcore_barrier()

      for b in range(NB):
        pltpu.async_copy(table.at[h].at[src_v.at[b]], bufs[b], semg[b])

      def body(t, carry):
        base = t * NB
        for b in range(NB):
          pltpu.make_async_copy(table.at[h].at[src_v.at[base + b]],
                                bufs[b], semg[b]).wait()
          pltpu.async_copy(bufs[b], acc.at[dst_v.at[base + b]], sems[b],
                           add=True)
        for b in range(NB):
          pltpu.make_async_copy(bufs[b], acc.at[dst_v.at[base + b]],
                                sems[b]).wait()
          pltpu.async_copy(table.at[h].at[src_v.at[base + NB + b]],
                           bufs[b], semg[b])
        return carry

      lax.fori_loop(0, k // NB - 1, body, 0)
      base = k - NB
      for b in range(NB):
        pltpu.make_async_copy(table.at[h].at[src_v.at[base + b]],
                              bufs[b], semg[b]).wait()
        pltpu.async_copy(bufs[b], acc.at[dst_v.at[base + b]], sems[b],
                         add=True)
      for b in range(NB):
        pltpu.make_async_copy(bufs[b], acc.at[dst_v.at[base + b]],
                              sems[b]).wait()
      plsc.subcore_barrier()
      pltpu.sync_copy(acc.at[rows], out.at[o].at[rows])

  return agg


def _tc_a(x_ref, w1_ref, deg_ref, g1_ref, dinv_ref):
  deg = deg_ref[0][:, 0:1] + deg_ref[1][:, 0:1] + 1.0
  dinv = lax.rsqrt(deg)
  h = jnp.dot(x_ref[...], w1_ref[...], preferred_element_type=_f32)
  g = (h * dinv).astype(jnp.bfloat16)
  for q in range(2):
    g1_ref[q] = g[:, q * 128:(q + 1) * 128]
  dinv_ref[...] = dinv


def _tc_b(s1_ref, g1_ref, dinv_ref, b1_ref, w2_ref, g2_ref):
  dinv = dinv_ref[...]
  z = jnp.concatenate(
      [s1_ref[q][...].astype(_f32) + g1_ref[q][...].astype(_f32)
       for q in range(2)], axis=1)
  z = jnp.maximum(z * dinv + b1_ref[...], 0.0)
  h2 = jnp.dot(z, w2_ref[...], preferred_element_type=_f32)
  g2_ref[...] = (h2 * dinv).astype(jnp.bfloat16)


def _tc_c(s2_ref, g2_ref, dinv_ref, b2_ref, out_ref):
  z = ((s2_ref[0][...].astype(_f32) + s2_ref[1][...].astype(_f32)
        + g2_ref[...].astype(_f32)) * dinv_ref[...] + b2_ref[...])
  mask = lax.broadcasted_iota(jnp.int32, (1, DOP), 1) < D_OUT
  zm = jnp.where(mask, z, -jnp.inf)
  mx = jnp.max(zm, axis=1, keepdims=True)
  ex = jnp.where(mask, jnp.exp(z - mx), 0.0)
  lse = jnp.log(jnp.sum(ex, axis=1, keepdims=True)) + mx
  out_ref[...] = z - lse


def kernel(x, edge_index, W1, b1, W2, b2):
  x_pad = jnp.zeros((NPAD, D_IN), _f32).at[:N].set(x)
  pad = jnp.full((EPAD - E,), N, jnp.int32)
  srcr = jnp.concatenate([edge_index[0], pad]).reshape(EROWS, 128)
  dstr = jnp.concatenate([edge_index[1], pad]).reshape(EROWS, 128)
  w2p = jnp.zeros((D_HID, DOP), _f32).at[:, :D_OUT].set(W2)
  b1r = b1.reshape(1, D_HID)
  b2p = jnp.zeros((1, DOP), _f32).at[0, :D_OUT].set(b2)
  ones16 = jnp.ones((128, 16), _f32)
  z16 = jnp.zeros((NPAD, 16), _f32)

  deg2 = _sc_deg()(dstr, ones16, z16)

  g1, dinv = pl.pallas_call(
      _tc_a,
      grid=(GRID_M,),
      in_specs=[
          pl.BlockSpec((BM, D_IN), lambda m: (m, 0)),
          pl.BlockSpec((D_IN, D_HID), lambda m: (0, 0)),
          pl.BlockSpec((2, BM, 16), lambda m: (0, m, 0)),
      ],
      out_specs=[
          pl.BlockSpec((2, BM, 128), lambda m: (0, m, 0)),
          pl.BlockSpec((BM, 1), lambda m: (m, 0)),
      ],
      out_shape=[
          jax.ShapeDtypeStruct((2, NPAD, 128), jnp.bfloat16),
          jax.ShapeDtypeStruct((NPAD, 1), _f32),
      ],
  )(x_pad, W1, deg2)

  z64b = jnp.zeros((NPAD, 64), jnp.bfloat16)
  z128b = jnp.zeros((NPAD, 128), jnp.bfloat16)
  s1 = _sc_agg(128, 2, K1, True, jnp.bfloat16)(g1, srcr, dstr, z128b)

  g2 = pl.pallas_call(
      _tc_b,
      grid=(GRID_M,),
      in_specs=[
          pl.BlockSpec((2, BM, 128), lambda m: (0, m, 0)),
          pl.BlockSpec((2, BM, 128), lambda m: (0, m, 0)),
          pl.BlockSpec((BM, 1), lambda m: (m, 0)),
          pl.BlockSpec((1, D_HID), lambda m: (0, 0)),
          pl.BlockSpec((D_HID, DOP), lambda m: (0, 0)),
      ],
      out_specs=pl.BlockSpec((BM, DOP), lambda m: (m, 0)),
      out_shape=jax.ShapeDtypeStruct((NPAD, DOP), jnp.bfloat16),
  )(s1, g1, dinv, b1r, w2p)

  s2 = _sc_agg(DOP, 1, K2, False, jnp.bfloat16)(g2[None], srcr, dstr, z64b)

  out_full = pl.pallas_call(
      _tc_c,
      grid=(GRID_M,),
      in_specs=[
          pl.BlockSpec((2, BM, DOP), lambda m: (0, m, 0)),
          pl.BlockSpec((BM, DOP), lambda m: (m, 0)),
          pl.BlockSpec((BM, 1), lambda m: (m, 0)),
          pl.BlockSpec((1, DOP), lambda m: (0, 0)),
      ],
      out_specs=pl.BlockSpec((BM, DOP), lambda m: (m, 0)),
      out_shape=jax.ShapeDtypeStruct((NPAD, DOP), _f32),
  )(s2, g2, dinv, b2p)

  return out_full[:N, :D_OUT]


# speedup vs baseline: 1.1980x; 1.0305x over previous
import functools

import jax
import jax.numpy as jnp
from jax import lax
from jax.experimental import pallas as pl
from jax.experimental.pallas import tpu as pltpu
from jax.experimental.pallas import tpu_sc as plsc

N = 10000
NPAD = 10240
E = 320000
EPAD = 327680
EROWS = EPAD // 128
D_IN = 128
D_HID = 256
D_OUT = 40
DOP = 64
NC, NS = 2, 16
RPT = NPAD // NS
K1 = EROWS // NS
K2 = EROWS // (NC * NS)
BM = 1024
GRID_M = NPAD // BM

_f32 = jnp.float32


def _mesh():
  return plsc.VectorSubcoreMesh(core_axis_name="c", subcore_axis_name="s")


def _sc_deg():
  @functools.partial(
      pl.kernel,
      out_type=jax.ShapeDtypeStruct((2, NPAD, 16), _f32),
      mesh=_mesh(),
      compiler_params=pltpu.CompilerParams(use_tc_tiling_on_sc=False),
      scratch_types=[
          pltpu.VMEM((K2, 128), jnp.int32),
          pltpu.VMEM((128, 16), _f32),
          pltpu.VMEM_SHARED((NPAD, 16), _f32),
      ],
  )
  def deg(dstr, ones, zeros, out, dst_v, ones_v, acc):
    c = lax.axis_index("c")
    s = lax.axis_index("s")
    rows = pl.ds(s * RPT, RPT)
    pltpu.sync_copy(zeros.at[rows], acc.at[rows])
    pltpu.sync_copy(ones, ones_v)
    row0 = (c * NS + s) * K2
    pltpu.sync_copy(dstr.at[pl.ds(row0, K2)], dst_v)
    plsc.subcore_barrier()

    def body(j, carry):
      pltpu.sync_copy(ones_v, acc.at[dst_v.at[j]], add=True)
      return carry

    lax.fori_loop(0, K2, body, 0)
    plsc.subcore_barrier()
    pltpu.sync_copy(acc.at[rows], out.at[c].at[rows])

  return deg


def _sc_agg(D, H, k, feature_split, dtype=_f32, dup_table=False):
  out_slabs = H if feature_split else 2
  passes = H // 2 if feature_split else 1
  NB = 8 if (dtype == jnp.bfloat16 and D <= 64) else 4
  assert k % NB == 0

  @functools.partial(
      pl.kernel,
      out_type=jax.ShapeDtypeStruct((out_slabs, NPAD, D), dtype),
      mesh=_mesh(),
      compiler_params=pltpu.CompilerParams(use_tc_tiling_on_sc=False),
      scratch_types=[
          pltpu.VMEM((k, 128), jnp.int32),
          pltpu.VMEM((k, 128), jnp.int32),
      ] + [pltpu.VMEM((128, D), dtype)] * NB
        + [pltpu.SemaphoreType.DMA] * (2 * NB)
      + [pltpu.VMEM_SHARED((NPAD, D), dtype)],
  )
  def agg(table, srcr, dstr, zeros, out, src_v, dst_v, *rest):
    bufs = rest[:NB]
    semg = rest[NB:2 * NB]
    sems = rest[2 * NB:3 * NB]
    acc = rest[3 * NB]
    c = lax.axis_index("c")
    s = lax.axis_index("s")
    rows = pl.ds(s * RPT, RPT)
    if feature_split:
      row0 = s * k
    else:
      row0 = (c * NS + s) * k
    pltpu.sync_copy(srcr.at[pl.ds(row0, k)], src_v)
    pltpu.sync_copy(dstr.at[pl.ds(row0, k)], dst_v)

    for p in range(passes):
      h = c * passes + p if feature_split else (c if dup_table else 0)
      o = h if feature_split else c
      pltpu.sync_copy(zeros.at[rows], acc.at[rows])
      plsc.subcore_barrier()

      for b in range(NB):
        pltpu.async_copy(table.at[h].at[src_v.at[b]], bufs[b], semg[b])

      def body(t, carry):
        base = t * NB
        for b in range(NB):
          pltpu.make_async_copy(table.at[h].at[src_v.at[base + b]],
                                bufs[b], semg[b]).wait()
          pltpu.async_copy(bufs[b], acc.at[dst_v.at[base + b]], sems[b],
                           add=True)
        for b in range(NB):
          pltpu.make_async_copy(bufs[b], acc.at[dst_v.at[base + b]],
                                sems[b]).wait()
          pltpu.async_copy(table.at[h].at[src_v.at[base + NB + b]],
                           bufs[b], semg[b])
        return carry

      lax.fori_loop(0, k // NB - 1, body, 0)
      base = k - NB
      for b in range(NB):
        pltpu.make_async_copy(table.at[h].at[src_v.at[base + b]],
                              bufs[b], semg[b]).wait()
        pltpu.async_copy(bufs[b], acc.at[dst_v.at[base + b]], sems[b],
                         add=True)
      for b in range(NB):
        pltpu.make_async_copy(bufs[b], acc.at[dst_v.at[base + b]],
                              sems[b]).wait()
      plsc.subcore_barrier()
      pltpu.sync_copy(acc.at[rows], out.at[o].at[rows])

  return agg


def _tc_a(x_ref, w1_ref, deg_ref, g1_ref, dinv_ref):
  deg = deg_ref[0][:, 0:1] + deg_ref[1][:, 0:1] + 1.0
  dinv = lax.rsqrt(deg)
  h = jnp.dot(x_ref[...], w1_ref[...], preferred_element_type=_f32)
  g = (h * dinv).astype(jnp.bfloat16)
  for q in range(2):
    g1_ref[q] = g[:, q * 128:(q + 1) * 128]
  dinv_ref[...] = dinv


def _tc_b(s1_ref, g1_ref, dinv_ref, b1_ref, w2_ref, g2_ref):
  dinv = dinv_ref[...]
  z = jnp.concatenate(
      [s1_ref[q][...].astype(_f32) + g1_ref[q][...].astype(_f32)
       for q in range(2)], axis=1)
  z = jnp.maximum(z * dinv + b1_ref[...], 0.0)
  h2 = jnp.dot(z, w2_ref[...], preferred_element_type=_f32)
  g2_ref[...] = (h2 * dinv).astype(jnp.bfloat16)


def _tc_c(s2_ref, g2_ref, dinv_ref, b2_ref, out_ref):
  z = ((s2_ref[0][...].astype(_f32) + s2_ref[1][...].astype(_f32)
        + g2_ref[...].astype(_f32)) * dinv_ref[...] + b2_ref[...])
  mask = lax.broadcasted_iota(jnp.int32, (1, DOP), 1) < D_OUT
  zm = jnp.where(mask, z, -jnp.inf)
  mx = jnp.max(zm, axis=1, keepdims=True)
  ex = jnp.where(mask, jnp.exp(z - mx), 0.0)
  lse = jnp.log(jnp.sum(ex, axis=1, keepdims=True)) + mx
  out_ref[...] = z - lse


def kernel(x, edge_index, W1, b1, W2, b2):
  x_pad = jnp.zeros((NPAD, D_IN), _f32).at[:N].set(x)
  pad = jnp.full((EPAD - E,), N, jnp.int32)
  srcr = jnp.concatenate([edge_index[0], pad]).reshape(EROWS, 128)
  dstr = jnp.concatenate([edge_index[1], pad]).reshape(EROWS, 128)
  w2p = jnp.zeros((D_HID, DOP), _f32).at[:, :D_OUT].set(W2)
  b1r = b1.reshape(1, D_HID)
  b2p = jnp.zeros((1, DOP), _f32).at[0, :D_OUT].set(b2)
  ones16 = jnp.ones((128, 16), _f32)
  z16 = jnp.zeros((NPAD, 16), _f32)

  deg2 = _sc_deg()(dstr, ones16, z16)

  g1, dinv = pl.pallas_call(
      _tc_a,
      grid=(GRID_M,),
      in_specs=[
          pl.BlockSpec((BM, D_IN), lambda m: (m, 0)),
          pl.BlockSpec((D_IN, D_HID), lambda m: (0, 0)),
          pl.BlockSpec((2, BM, 16), lambda m: (0, m, 0)),
      ],
      out_specs=[
          pl.BlockSpec((2, BM, 128), lambda m: (0, m, 0)),
          pl.BlockSpec((BM, 1), lambda m: (m, 0)),
      ],
      out_shape=[
          jax.ShapeDtypeStruct((2, NPAD, 128), jnp.bfloat16),
          jax.ShapeDtypeStruct((NPAD, 1), _f32),
      ],
  )(x_pad, W1, deg2)

  z64b = jnp.zeros((NPAD, 64), jnp.bfloat16)
  z128b = jnp.zeros((NPAD, 128), jnp.bfloat16)
  s1 = _sc_agg(128, 2, K1, True, jnp.bfloat16)(g1, srcr, dstr, z128b)

  g2 = pl.pallas_call(
      _tc_b,
      grid=(GRID_M,),
      in_specs=[
          pl.BlockSpec((2, BM, 128), lambda m: (0, m, 0)),
          pl.BlockSpec((2, BM, 128), lambda m: (0, m, 0)),
          pl.BlockSpec((BM, 1), lambda m: (m, 0)),
          pl.BlockSpec((1, D_HID), lambda m: (0, 0)),
          pl.BlockSpec((D_HID, DOP), lambda m: (0, 0)),
      ],
      out_specs=pl.BlockSpec((BM, DOP), lambda m: (m, 0)),
      out_shape=jax.ShapeDtypeStruct((NPAD, DOP), jnp.bfloat16),
  )(s1, g1, dinv, b1r, w2p)

  s2 = _sc_agg(DOP, 1, K2, False, jnp.bfloat16)(g2[None], srcr, dstr, z64b)

  out_full = pl.pallas_call(
      _tc_c,
      grid=(GRID_M,),
      in_specs=[
          pl.BlockSpec((2, BM, DOP), lambda m: (0, m, 0)),
          pl.BlockSpec((BM, DOP), lambda m: (m, 0)),
          pl.BlockSpec((BM, 1), lambda m: (m, 0)),
          pl.BlockSpec((1, DOP), lambda m: (0, 0)),
      ],
      out_specs=pl.BlockSpec((BM, DOP), lambda m: (m, 0)),
      out_shape=jax.ShapeDtypeStruct((NPAD, DOP), _f32),
  )(s2, g2, dinv, b2p)

  return out_full[:N, :D_OUT]
